# Initial kernel scaffold; baseline (speedup 1.0000x reference)
import functools

import jax
import jax.numpy as jnp
from jax import lax
from jax.experimental import pallas as pl
from jax.experimental.pallas import tpu as pltpu
from jax.experimental.pallas import tpu_sc as plsc

N, E, D, H, C = 10000, 320000, 128, 128, 64

NC, NS = 2, 16
NW = NC * NS
E_PER_TILE = E // NW
CHUNK = 80
N_CHUNKS = E_PER_TILE // CHUNK
N_PAD = 10240
ROWS_PER_SUB = N_PAD // NS
DEG_W = 16

_sc_mesh = plsc.VectorSubcoreMesh(
    core_axis_name="c", subcore_axis_name="s", num_cores=NC, num_subcores=NS
)


@functools.partial(
    pl.kernel,
    out_type=[
        jax.ShapeDtypeStruct((NC, N_PAD, D), jnp.float32),
        jax.ShapeDtypeStruct((NC, N_PAD, DEG_W), jnp.float32),
    ],
    mesh=_sc_mesh,
    scratch_types=[
        pltpu.VMEM((CHUNK,), jnp.int32),
        pltpu.VMEM((CHUNK,), jnp.int32),
        pltpu.VMEM((CHUNK, D), jnp.float32),
        pltpu.VMEM((CHUNK, DEG_W), jnp.float32),
        pltpu.VMEM_SHARED((N_PAD, D), jnp.float32),
        pltpu.VMEM_SHARED((N_PAD, DEG_W), jnp.float32),
    ],
)
def _sc_aggregate(
    x_hbm, src_hbm, dst_hbm, zeros_d_hbm, zeros_w_hbm, ones_hbm,
    agg_out, deg_out,
    src_v, dst_v, rows_v, ones_v, agg_sh, deg_sh,
):
    cid = lax.axis_index("c")
    sid = lax.axis_index("s")

    r0 = sid * ROWS_PER_SUB
    pltpu.sync_copy(
        zeros_d_hbm.at[pl.ds(r0, ROWS_PER_SUB)],
        agg_sh.at[pl.ds(r0, ROWS_PER_SUB)],
    )
    pltpu.sync_copy(
        zeros_w_hbm.at[pl.ds(r0, ROWS_PER_SUB)],
        deg_sh.at[pl.ds(r0, ROWS_PER_SUB)],
    )
    pltpu.sync_copy(ones_hbm, ones_v)
    plsc.subcore_barrier()

    base = (cid * NS + sid) * E_PER_TILE

    @pl.loop(0, N_CHUNKS)
    def _(g):
        off = base + g * CHUNK
        pltpu.sync_copy(src_hbm.at[pl.ds(off, CHUNK)], src_v)
        pltpu.sync_copy(dst_hbm.at[pl.ds(off, CHUNK)], dst_v)
        pltpu.sync_copy(x_hbm.at[src_v], rows_v)
        pltpu.sync_copy(rows_v, agg_sh.at[dst_v], add=True)
        pltpu.sync_copy(ones_v, deg_sh.at[dst_v], add=True)

    plsc.subcore_barrier()

    pltpu.sync_copy(
        agg_sh.at[pl.ds(r0, ROWS_PER_SUB)],
        agg_out.at[cid, pl.ds(r0, ROWS_PER_SUB)],
    )
    pltpu.sync_copy(
        deg_sh.at[pl.ds(r0, ROWS_PER_SUB)],
        deg_out.at[cid, pl.ds(r0, ROWS_PER_SUB)],
    )


def _tc_body(
    agg_ref, deg_ref, x_ref, wl_ref, bl_ref, wr_ref, wlin_ref, blin_ref,
    out_ref, lab_ref,
):
    agg = agg_ref[0] + agg_ref[1]
    deg = deg_ref[0, :, 0] + deg_ref[1, :, 0]
    agg = agg / jnp.maximum(deg, 1.0)[:, None]
    h = (
        jnp.dot(agg, wl_ref[...], preferred_element_type=jnp.float32)
        + bl_ref[...]
        + jnp.dot(x_ref[...], wr_ref[...], preferred_element_type=jnp.float32)
    )
    h = jnp.maximum(h, 0.0)
    logits = (
        jnp.dot(h, wlin_ref[...], preferred_element_type=jnp.float32)
        + blin_ref[...]
    )
    m = jnp.max(logits, axis=1, keepdims=True)
    lse = jnp.log(jnp.sum(jnp.exp(logits - m), axis=1, keepdims=True)) + m
    o = logits - lse
    out_ref[...] = o
    lab_ref[...] = jnp.argmax(o, axis=1).astype(jnp.int32)


_TC_R = 1024


def _tc_head(agg_parts, deg_parts, x_pad, W_l, b_l2, W_r, W_lin, b_lin2):
    grid = (N_PAD // _TC_R,)
    return pl.pallas_call(
        _tc_body,
        grid=grid,
        in_specs=[
            pl.BlockSpec((NC, _TC_R, D), lambda i: (0, i, 0)),
            pl.BlockSpec((NC, _TC_R, DEG_W), lambda i: (0, i, 0)),
            pl.BlockSpec((_TC_R, D), lambda i: (i, 0)),
            pl.BlockSpec((D, H), lambda i: (0, 0)),
            pl.BlockSpec((1, H), lambda i: (0, 0)),
            pl.BlockSpec((D, H), lambda i: (0, 0)),
            pl.BlockSpec((H, C), lambda i: (0, 0)),
            pl.BlockSpec((1, C), lambda i: (0, 0)),
        ],
        out_specs=[
            pl.BlockSpec((_TC_R, C), lambda i: (i, 0)),
            pl.BlockSpec((_TC_R,), lambda i: (i,)),
        ],
        out_shape=[
            jax.ShapeDtypeStruct((N_PAD, C), jnp.float32),
            jax.ShapeDtypeStruct((N_PAD,), jnp.int32),
        ],
    )(agg_parts, deg_parts, x_pad, W_l, b_l2, W_r, W_lin, b_lin2)


def kernel(x, edge_index, W_l, b_l, W_r, W_lin, b_lin):
    src = edge_index[0]
    dst = edge_index[1]
    zeros_d = jnp.zeros((N_PAD, D), jnp.float32)
    zeros_w = jnp.zeros((N_PAD, DEG_W), jnp.float32)
    ones_c = jnp.ones((CHUNK, DEG_W), jnp.float32)
    agg_parts, deg_parts = _sc_aggregate(x, src, dst, zeros_d, zeros_w, ones_c)
    x_pad = jnp.concatenate([x, jnp.zeros((N_PAD - N, D), x.dtype)], axis=0)
    out_pad, lab_pad = _tc_head(
        agg_parts, deg_parts, x_pad,
        W_l, b_l.reshape(1, H), W_r, W_lin, b_lin.reshape(1, C),
    )
    return lab_pad[:N], out_pad[:N]


# speedup vs baseline: 6.0634x; 6.0634x over previous
import dataclasses
import functools

import jax
import jax.numpy as jnp
from jax import lax
from jax.experimental import pallas as pl
from jax.experimental.pallas import tpu as pltpu
from jax.experimental.pallas import tpu_sc as plsc

N, E, D, H, C = 10000, 320000, 128, 128, 64

NC, NS = 2, 16
NW = NC * NS
E_PER_TILE = E // NW
CHUNK = 80
N_CHUNKS = E_PER_TILE // CHUNK
N_PAD = 10240
ROWS_PER_SUB = N_PAD // NS
DEG_W = 16

_sc_mesh = plsc.VectorSubcoreMesh(
    core_axis_name="c", subcore_axis_name="s", num_cores=NC, num_subcores=NS
)

_sc_params = pltpu.CompilerParams()
if "needs_layout_passes" in pltpu.CompilerParams.__dataclass_fields__:
    _sc_params = dataclasses.replace(_sc_params, needs_layout_passes=False)


@functools.partial(
    pl.kernel,
    out_type=[
        jax.ShapeDtypeStruct((NC, N_PAD, D), jnp.float32),
        jax.ShapeDtypeStruct((NC, N_PAD), jnp.float32),
    ],
    mesh=_sc_mesh,
    scratch_types=[
        pltpu.VMEM((CHUNK,), jnp.int32),
        pltpu.VMEM((CHUNK,), jnp.int32),
        pltpu.VMEM((CHUNK, D), jnp.float32),
        pltpu.VMEM((N_PAD,), jnp.float32),
        pltpu.VMEM((ROWS_PER_SUB,), jnp.float32),
        pltpu.VMEM((ROWS_PER_SUB,), jnp.float32),
        pltpu.VMEM_SHARED((N_PAD, D), jnp.float32),
        pltpu.VMEM_SHARED((NS, N_PAD), jnp.float32),
        pltpu.SemaphoreType.DMA,
    ],
    compiler_params=_sc_params,
)
def _sc_aggregate(
    x_hbm, src_hbm, dst_hbm, zeros_d_hbm,
    agg_out, deg_out,
    src_v, dst_v, rows_v, deg_v, acc_v, tmp_v, agg_sh, deg_sh, sem,
):
    cid = lax.axis_index("c")
    sid = lax.axis_index("s")
    r0 = sid * ROWS_PER_SUB

    pltpu.sync_copy(
        zeros_d_hbm.at[pl.ds(r0, ROWS_PER_SUB)],
        agg_sh.at[pl.ds(r0, ROWS_PER_SUB)],
    )

    @pl.loop(0, N_PAD // 16)
    def _(i):
        deg_v[pl.ds(i * 16, 16)] = jnp.zeros((16,), jnp.float32)

    plsc.subcore_barrier()

    base = (cid * NS + sid) * E_PER_TILE
    one16 = jnp.ones((16,), jnp.float32)

    @pl.loop(0, N_CHUNKS)
    def _(g):
        off = base + g * CHUNK
        pltpu.sync_copy(src_hbm.at[pl.ds(off, CHUNK)], src_v)
        pltpu.sync_copy(dst_hbm.at[pl.ds(off, CHUNK)], dst_v)
        pltpu.sync_copy(x_hbm.at[src_v], rows_v)
        pltpu.sync_copy(rows_v, agg_sh.at[dst_v], add=True)
        for k in range(CHUNK // 16):
            idx16 = dst_v[pl.ds(k * 16, 16)]
            plsc.addupdate_scatter(deg_v, [idx16], one16)

    pltpu.sync_copy(deg_v, deg_sh.at[sid])
    plsc.subcore_barrier()

    pltpu.sync_copy(deg_sh.at[0, pl.ds(r0, ROWS_PER_SUB)], acc_v)

    @pl.loop(1, NS)
    def _(t):
        pltpu.sync_copy(deg_sh.at[t, pl.ds(r0, ROWS_PER_SUB)], tmp_v)

        @pl.loop(0, ROWS_PER_SUB // 16)
        def _(i):
            sl = pl.ds(i * 16, 16)
            acc_v[sl] = acc_v[sl] + tmp_v[sl]

    pltpu.sync_copy(
        agg_sh.at[pl.ds(r0, ROWS_PER_SUB)],
        agg_out.at[cid, pl.ds(r0, ROWS_PER_SUB)],
    )
    pltpu.sync_copy(acc_v, deg_out.at[cid, pl.ds(r0, ROWS_PER_SUB)])


def _tc_body(
    agg_ref, deg_ref, x_ref, wl_ref, bl_ref, wr_ref, wlin_ref, blin_ref,
    out_ref, lab_ref,
):
    agg = agg_ref[0] + agg_ref[1]
    deg = deg_ref[0] + deg_ref[1]
    agg = agg / jnp.maximum(deg, 1.0)[:, None]
    h = (
        jnp.dot(agg, wl_ref[...], preferred_element_type=jnp.float32)
        + bl_ref[...]
        + jnp.dot(x_ref[...], wr_ref[...], preferred_element_type=jnp.float32)
    )
    h = jnp.maximum(h, 0.0)
    logits = (
        jnp.dot(h, wlin_ref[...], preferred_element_type=jnp.float32)
        + blin_ref[...]
    )
    m = jnp.max(logits, axis=1, keepdims=True)
    lse = jnp.log(jnp.sum(jnp.exp(logits - m), axis=1, keepdims=True)) + m
    o = logits - lse
    out_ref[...] = o
    lab_ref[...] = jnp.argmax(o, axis=1).astype(jnp.int32)


_TC_R = 1024


def _tc_head(agg_parts, deg_parts, x_pad, W_l, b_l2, W_r, W_lin, b_lin2):
    grid = (N_PAD // _TC_R,)
    return pl.pallas_call(
        _tc_body,
        grid=grid,
        in_specs=[
            pl.BlockSpec((NC, _TC_R, D), lambda i: (0, i, 0)),
            pl.BlockSpec((NC, _TC_R), lambda i: (0, i)),
            pl.BlockSpec((_TC_R, D), lambda i: (i, 0)),
            pl.BlockSpec((D, H), lambda i: (0, 0)),
            pl.BlockSpec((1, H), lambda i: (0, 0)),
            pl.BlockSpec((D, H), lambda i: (0, 0)),
            pl.BlockSpec((H, C), lambda i: (0, 0)),
            pl.BlockSpec((1, C), lambda i: (0, 0)),
        ],
        out_specs=[
            pl.BlockSpec((_TC_R, C), lambda i: (i, 0)),
            pl.BlockSpec((_TC_R,), lambda i: (i,)),
        ],
        out_shape=[
            jax.ShapeDtypeStruct((N_PAD, C), jnp.float32),
            jax.ShapeDtypeStruct((N_PAD,), jnp.int32),
        ],
    )(agg_parts, deg_parts, x_pad, W_l, b_l2, W_r, W_lin, b_lin2)


def kernel(x, edge_index, W_l, b_l, W_r, W_lin, b_lin):
    src = edge_index[0]
    dst = edge_index[1]
    zeros_d = jnp.zeros((N_PAD, D), jnp.float32)
    agg_parts, deg_parts = _sc_aggregate(x, src, dst, zeros_d)
    x_pad = jnp.concatenate([x, jnp.zeros((N_PAD - N, D), x.dtype)], axis=0)
    out_pad, lab_pad = _tc_head(
        agg_parts, deg_parts, x_pad,
        W_l, b_l.reshape(1, H), W_r, W_lin, b_lin.reshape(1, C),
    )
    return lab_pad[:N], out_pad[:N]


# speedup vs baseline: 10.2462x; 1.6898x over previous
import dataclasses
import functools

import jax
import jax.numpy as jnp
from jax import lax
from jax.experimental import pallas as pl
from jax.experimental.pallas import tpu as pltpu
from jax.experimental.pallas import tpu_sc as plsc

N, E, D, H, C = 10000, 320000, 128, 128, 64

NC, NS = 2, 16
NW = NC * NS
E_PER_TILE = E // NW
CHUNK = 80
N_CHUNKS = E_PER_TILE // CHUNK
N_ACC = 10240
ROWS_PER_SUB = N_ACC // NS

_sc_mesh = plsc.VectorSubcoreMesh(
    core_axis_name="c", subcore_axis_name="s", num_cores=NC, num_subcores=NS
)

_sc_params = pltpu.CompilerParams()
if "needs_layout_passes" in pltpu.CompilerParams.__dataclass_fields__:
    _sc_params = dataclasses.replace(_sc_params, needs_layout_passes=False)


@functools.partial(
    pl.kernel,
    out_type=[
        jax.ShapeDtypeStruct((NC, N_ACC, D), jnp.float32),
        jax.ShapeDtypeStruct((NW * N,), jnp.float32),
    ],
    mesh=_sc_mesh,
    scratch_types=[
        pltpu.VMEM((E_PER_TILE,), jnp.int32),
        pltpu.VMEM((CHUNK,), jnp.int32),
        pltpu.VMEM((CHUNK,), jnp.int32),
        pltpu.VMEM((CHUNK, D), jnp.float32),
        pltpu.VMEM((CHUNK, D), jnp.float32),
        pltpu.VMEM((N,), jnp.float32),
        pltpu.VMEM_SHARED((N_ACC, D), jnp.float32),
        pltpu.SemaphoreType.DMA,
        pltpu.SemaphoreType.DMA,
        pltpu.SemaphoreType.DMA,
        pltpu.SemaphoreType.DMA,
    ],
    compiler_params=_sc_params,
)
def _sc_aggregate(
    x_hbm, src_hbm, dst_hbm, zeros_d_hbm,
    agg_out, deg_out,
    src_v, dstb0, dstb1, rows0_v, rows1_v, deg_v, agg_sh,
    semr0, semr1, semi0, semi1,
):
    cid = lax.axis_index("c")
    sid = lax.axis_index("s")
    r0 = sid * ROWS_PER_SUB
    wid = cid * NS + sid
    ebase = wid * E_PER_TILE

    pltpu.sync_copy(src_hbm.at[pl.ds(ebase, E_PER_TILE)], src_v)

    pltpu.sync_copy(
        zeros_d_hbm.at[pl.ds(r0, ROWS_PER_SUB)],
        agg_sh.at[pl.ds(r0, ROWS_PER_SUB)],
    )

    @pl.loop(0, N // 16)
    def _(i):
        deg_v[pl.ds(i * 16, 16)] = jnp.zeros((16,), jnp.float32)

    plsc.subcore_barrier()

    one16 = jnp.ones((16,), jnp.float32)

    def start_chunk(g, rows_v, dstb, semr, semi):
        idx = src_v.at[pl.ds(g * CHUNK, CHUNK)]
        pltpu.async_copy(x_hbm.at[idx], rows_v, semr)
        pltpu.async_copy(dst_hbm.at[pl.ds(ebase + g * CHUNK, CHUNK)], dstb, semi)

    def wait_chunk(g, rows_v, dstb, semr, semi):
        idx = src_v.at[pl.ds(g * CHUNK, CHUNK)]
        pltpu.make_async_copy(x_hbm.at[idx], rows_v, semr).wait()
        pltpu.make_async_copy(
            dst_hbm.at[pl.ds(ebase + g * CHUNK, CHUNK)], dstb, semi
        ).wait()

    def scatter(rows_v, dstb):
        pltpu.sync_copy(rows_v, agg_sh.at[dstb], add=True)
        for k in range(CHUNK // 16):
            idx16 = dstb[pl.ds(k * 16, 16)]
            plsc.addupdate_scatter(deg_v, [idx16], one16)

    start_chunk(0, rows0_v, dstb0, semr0, semi0)

    @pl.loop(0, (N_CHUNKS - 1) // 2)
    def _(p):
        i0 = 2 * p
        wait_chunk(i0, rows0_v, dstb0, semr0, semi0)
        start_chunk(i0 + 1, rows1_v, dstb1, semr1, semi1)
        scatter(rows0_v, dstb0)
        wait_chunk(i0 + 1, rows1_v, dstb1, semr1, semi1)
        start_chunk(i0 + 2, rows0_v, dstb0, semr0, semi0)
        scatter(rows1_v, dstb1)

    wait_chunk(N_CHUNKS - 1, rows0_v, dstb0, semr0, semi0)
    scatter(rows0_v, dstb0)

    plsc.subcore_barrier()

    pltpu.sync_copy(
        agg_sh.at[pl.ds(r0, ROWS_PER_SUB)],
        agg_out.at[cid, pl.ds(r0, ROWS_PER_SUB)],
    )
    pltpu.sync_copy(deg_v, deg_out.at[pl.ds(wid * N, N)])


def _tc_body(
    agg_ref, deg_ref, x_ref, wl_ref, bl_ref, wr_ref, wlin_ref, blin_ref,
    out_ref, lab_ref,
):
    agg = agg_ref[0] + agg_ref[1]
    deg = jnp.sum(deg_ref[...], axis=1)
    agg = agg / jnp.maximum(deg, 1.0)[:, None]
    h = (
        jnp.dot(agg, wl_ref[...], preferred_element_type=jnp.float32)
        + bl_ref[...]
        + jnp.dot(x_ref[...], wr_ref[...], preferred_element_type=jnp.float32)
    )
    h = jnp.maximum(h, 0.0)
    logits = (
        jnp.dot(h, wlin_ref[...], preferred_element_type=jnp.float32)
        + blin_ref[...]
    )
    m = jnp.max(logits, axis=1, keepdims=True)
    lse = jnp.log(jnp.sum(jnp.exp(logits - m), axis=1, keepdims=True)) + m
    o = logits - lse
    out_ref[...] = o
    lab_ref[...] = jnp.argmax(o, axis=1).astype(jnp.int32)[:, None]


_TC_R = 1000


def _tc_head(agg_parts, deg_t, x, W_l, b_l2, W_r, W_lin, b_lin2):
    grid = (N // _TC_R,)
    return pl.pallas_call(
        _tc_body,
        grid=grid,
        in_specs=[
            pl.BlockSpec((NC, _TC_R, D), lambda i: (0, i, 0)),
            pl.BlockSpec((_TC_R, NW), lambda i: (i, 0)),
            pl.BlockSpec((_TC_R, D), lambda i: (i, 0)),
            pl.BlockSpec((D, H), lambda i: (0, 0)),
            pl.BlockSpec((1, H), lambda i: (0, 0)),
            pl.BlockSpec((D, H), lambda i: (0, 0)),
            pl.BlockSpec((H, C), lambda i: (0, 0)),
            pl.BlockSpec((1, C), lambda i: (0, 0)),
        ],
        out_specs=[
            pl.BlockSpec((_TC_R, C), lambda i: (i, 0)),
            pl.BlockSpec((_TC_R, 1), lambda i: (i, 0)),
        ],
        out_shape=[
            jax.ShapeDtypeStruct((N, C), jnp.float32),
            jax.ShapeDtypeStruct((N, 1), jnp.int32),
        ],
    )(agg_parts, deg_t, x, W_l, b_l2, W_r, W_lin, b_lin2)


def kernel(x, edge_index, W_l, b_l, W_r, W_lin, b_lin):
    src = edge_index[0]
    dst = edge_index[1]
    zeros_d = jnp.zeros((N_ACC, D), jnp.float32)
    agg_parts, deg_flat = _sc_aggregate(x, src, dst, zeros_d)
    deg_t = deg_flat.reshape(NW, N).T
    out, lab2 = _tc_head(
        agg_parts, deg_t, x,
        W_l, b_l.reshape(1, H), W_r, W_lin, b_lin.reshape(1, C),
    )
    return lab2.reshape(N), out
